# baseline (speedup 1.0000x reference)
import functools

import jax
import jax.numpy as jnp
from jax import lax
from jax.experimental import pallas as pl
from jax.experimental.pallas import tpu as pltpu
from jax.experimental.pallas import tpu_sc as plsc

EMBED_DIM = 10
HIDDEN = 128
BATCH = 16384

SUP = 4
SUP_DIM = SUP * EMBED_DIM

NUM_CORES = 2
NUM_SUBCORES = 16
NUM_WORKERS = NUM_CORES * NUM_SUBCORES
B_PER_W = BATCH // NUM_WORKERS
CHUNK = 128
NCHUNK = B_PER_W // CHUNK
ROW_BLOCKS = BATCH // CHUNK


def _gather_body(did_hbm, aid_hbm, dtab_hbm, atab_hbm, outd_hbm, outa_hbm,
                 idxd_v, idxa_v, rowsd_v, rowsa_v, sem):
  wid = lax.axis_index("s") * NUM_CORES + lax.axis_index("c")
  base = wid * NCHUNK
  pltpu.sync_copy(did_hbm.at[pl.ds(base, NCHUNK)], idxd_v)
  pltpu.sync_copy(aid_hbm.at[pl.ds(base, NCHUNK)], idxa_v)
  copies = []
  for j in range(NCHUNK):
    copies.append(pltpu.async_copy(
        dtab_hbm.at[idxd_v.at[j]], rowsd_v.at[j], sem))
    copies.append(pltpu.async_copy(
        atab_hbm.at[idxa_v.at[j]], rowsa_v.at[j], sem))
  for c in copies:
    c.wait()
  pltpu.sync_copy(rowsd_v, outd_hbm.at[pl.ds(base, NCHUNK)])
  pltpu.sync_copy(rowsa_v, outa_hbm.at[pl.ds(base, NCHUNK)])


@functools.cache
def _sc_gather():
  return functools.partial(
      pl.kernel,
      out_type=[
          jax.ShapeDtypeStruct((ROW_BLOCKS, CHUNK, SUP_DIM), jnp.float32),
          jax.ShapeDtypeStruct((ROW_BLOCKS, CHUNK, SUP_DIM), jnp.float32),
      ],
      mesh=plsc.VectorSubcoreMesh(core_axis_name="c", subcore_axis_name="s",
                                  num_cores=NUM_CORES,
                                  num_subcores=NUM_SUBCORES),
      scratch_types=[
          pltpu.VMEM((NCHUNK, CHUNK), jnp.int32),
          pltpu.VMEM((NCHUNK, CHUNK), jnp.int32),
          pltpu.VMEM((NCHUNK, CHUNK, SUP_DIM), jnp.float32),
          pltpu.VMEM((NCHUNK, CHUNK, SUP_DIM), jnp.float32),
          pltpu.SemaphoreType.DMA,
      ],
      compiler_params=pltpu.CompilerParams(use_tc_tiling_on_sc=False),
  )(_gather_body)


def _mlp_body(dsup_ref, asup_ref, did_ref, aid_ref, w1d_ref, w1a_ref,
              b1_ref, w2_ref, b2_ref, o_ref):
  blk = dsup_ref.shape[0]
  grp = jax.lax.broadcasted_iota(jnp.int32, (blk, SUP_DIM), 1) // EMBED_DIM
  rd = (did_ref[...] & (SUP - 1)).reshape(blk, 1)
  ra = (aid_ref[...] & (SUP - 1)).reshape(blk, 1)
  xd = jnp.where(grp == rd, dsup_ref[...], 0.0)
  xa = jnp.where(grp == ra, asup_ref[...], 0.0)
  h = jnp.dot(xd, w1d_ref[...], preferred_element_type=jnp.float32)
  h = h + jnp.dot(xa, w1a_ref[...], preferred_element_type=jnp.float32)
  h = jnp.maximum(h + b1_ref[...], 0.0)
  o_ref[...] = jnp.sum(h * w2_ref[...], axis=1) + b2_ref[0, 0]


def _mlp(d_sup, a_sup, did, aid, w1d4, w1a4, b1r, w2r, b2r):
  blk = 2048
  grid = BATCH // blk
  return pl.pallas_call(
      _mlp_body,
      out_shape=jax.ShapeDtypeStruct((BATCH,), jnp.float32),
      grid=(grid,),
      in_specs=[
          pl.BlockSpec((blk, SUP_DIM), lambda i: (i, 0)),
          pl.BlockSpec((blk, SUP_DIM), lambda i: (i, 0)),
          pl.BlockSpec((blk,), lambda i: (i,)),
          pl.BlockSpec((blk,), lambda i: (i,)),
          pl.BlockSpec((SUP_DIM, HIDDEN), lambda i: (0, 0)),
          pl.BlockSpec((SUP_DIM, HIDDEN), lambda i: (0, 0)),
          pl.BlockSpec((1, HIDDEN), lambda i: (0, 0)),
          pl.BlockSpec((1, HIDDEN), lambda i: (0, 0)),
          pl.BlockSpec((1, 1), lambda i: (0, 0)),
      ],
      out_specs=pl.BlockSpec((blk,), lambda i: (i,)),
  )(d_sup, a_sup, did, aid, w1d4, w1a4, b1r, w2r, b2r)


def kernel(domain_id, author_id, domain_table, author_table, W1, b1, W2, b2):
  did = domain_id.astype(jnp.int32)
  aid = author_id.astype(jnp.int32)
  did_sup = (did >> 2).reshape(ROW_BLOCKS, CHUNK)
  aid_sup = (aid >> 2).reshape(ROW_BLOCKS, CHUNK)
  dtab4 = domain_table.reshape(-1, SUP_DIM)
  atab4 = author_table.reshape(-1, SUP_DIM)
  d_rows, a_rows = _sc_gather()(did_sup, aid_sup, dtab4, atab4)
  d_sup = d_rows.reshape(BATCH, SUP_DIM)
  a_sup = a_rows.reshape(BATCH, SUP_DIM)
  w1d4 = jnp.tile(W1[:EMBED_DIM], (SUP, 1))
  w1a4 = jnp.tile(W1[EMBED_DIM:], (SUP, 1))
  b1r = b1.reshape(1, HIDDEN)
  w2r = W2.reshape(1, HIDDEN)
  b2r = b2.reshape(1, 1)
  return _mlp(d_sup, a_sup, did, aid, w1d4, w1a4, b1r, w2r, b2r)
